# Initial kernel scaffold; baseline (speedup 1.0000x reference)
import jax
import jax.numpy as jnp
from jax.experimental import pallas as pl


def kernel(x, W1, g1, b1, W2, g2, b2, Wp, bp):
    raise NotImplementedError("write your pallas kernel here")


# speedup vs baseline: 8.5121x; 8.5121x over previous
import jax
import jax.numpy as jnp
from jax.experimental import pallas as pl

B, N, KNN = 64, 1024, 4
RB = 4
R = N // RB


def _bf16_dot(a, b):
    return jax.lax.dot_general(a.astype(jnp.bfloat16), b.astype(jnp.bfloat16),
                               (((1,), (1,)), ((), ())),
                               preferred_element_type=jnp.float32)


def _select4(d, gather_from):
    iota = jax.lax.broadcasted_iota(jnp.int32, (R, N), 1)
    gs = []
    for _ in range(KNN):
        m = jnp.min(d, axis=1, keepdims=True)
        eq = d == m
        idx = jnp.min(jnp.where(eq, iota, N), axis=1, keepdims=True)
        oh = iota == idx
        g = jax.lax.dot_general(oh.astype(jnp.float32), gather_from,
                                (((1,), (0,)), ((), ())),
                                precision=jax.lax.Precision.HIGHEST,
                                preferred_element_type=jnp.float32)
        gs.append(g)
        d = jnp.where(oh, jnp.inf, d)
    return gs


def _knn1_body(xf_ref, xr_ref, sqr_ref, sqj_ref, xj_ref):
    xf = xf_ref[0]
    xr = xr_ref[0]
    d = (sqr_ref[0] + sqj_ref[0]) - 2.0 * _bf16_dot(xr, xf)
    gs = _select4(d, xf)
    for s in range(KNN):
        xj_ref[0, :, s] = gs[s]


def _knn2_body(xf_ref, xr_ref, sqr_ref, sqj_ref, w_ref, h_ref):
    xf = xf_ref[0]
    xr = xr_ref[0]
    d = (sqr_ref[0] + sqj_ref[0]) - 2.0 * _bf16_dot(xr, xf)
    gs = _select4(d, xf)
    W = w_ref[...]
    for s in range(KNN):
        edge = jnp.concatenate([xr, gs[s] - xr], axis=1)
        h_ref[0, :, s] = jax.lax.dot_general(
            edge, W, (((1,), (0,)), ((), ())),
            preferred_element_type=jnp.float32)


def _finish_body(h_ref, mn_ref, sd_ref, ga_ref, be_ref, xm_ref,
                 wp_ref, bp_ref, out_ref):
    h = h_ref[0]
    hn = ((h - mn_ref[...]) / sd_ref[...]) * ga_ref[...] + be_ref[...]
    hn = jnp.clip(hn, 0.0, 6.0)
    x2 = jnp.max(hn, axis=1)
    x2m = jnp.max(x2, axis=0, keepdims=True)
    pooled = jnp.concatenate([xm_ref[0], x2m], axis=1)
    o = jax.lax.dot_general(pooled, wp_ref[...], (((1,), (0,)), ((), ())),
                            preferred_element_type=jnp.float32)
    out_ref[0] = jnp.clip(o + bp_ref[...], 0.0, 6.0)


def _knn1(x, sqc, sqr):
    return pl.pallas_call(
        _knn1_body,
        grid=(B, RB),
        in_specs=[
            pl.BlockSpec((1, N, 3), lambda b, rb: (b, 0, 0)),
            pl.BlockSpec((1, R, 3), lambda b, rb: (b, rb, 0)),
            pl.BlockSpec((1, R, 1), lambda b, rb: (b, rb, 0)),
            pl.BlockSpec((1, 1, N), lambda b, rb: (b, 0, 0)),
        ],
        out_specs=pl.BlockSpec((1, R, KNN, 3), lambda b, rb: (b, rb, 0, 0)),
        out_shape=jax.ShapeDtypeStruct((B, N, KNN, 3), jnp.float32),
    )(x, x, sqc, sqr)


def _knn2(x1, sqc, sqr, W2):
    return pl.pallas_call(
        _knn2_body,
        grid=(B, RB),
        in_specs=[
            pl.BlockSpec((1, N, 64), lambda b, rb: (b, 0, 0)),
            pl.BlockSpec((1, R, 64), lambda b, rb: (b, rb, 0)),
            pl.BlockSpec((1, R, 1), lambda b, rb: (b, rb, 0)),
            pl.BlockSpec((1, 1, N), lambda b, rb: (b, 0, 0)),
            pl.BlockSpec((128, 64), lambda b, rb: (0, 0)),
        ],
        out_specs=pl.BlockSpec((1, R, KNN, 64), lambda b, rb: (b, rb, 0, 0)),
        out_shape=jax.ShapeDtypeStruct((B, N, KNN, 64), jnp.float32),
    )(x1, x1, sqc, sqr, W2)


@jax.jit
def kernel(x, W1, g1, b1, W2, g2, b2, Wp, bp):
    sq1 = jnp.sum(x * x, axis=-1)
    xj = _knn1(x, sq1[:, :, None], sq1[:, None, :])

    xi = jnp.broadcast_to(x[:, :, None, :], xj.shape)
    edge1 = jnp.concatenate([xi, xj - xi], axis=-1)
    h1 = edge1 @ W1
    mean1 = jnp.mean(h1, axis=(0, 1, 2))
    var1 = jnp.var(h1, axis=(0, 1, 2))
    hn1 = (h1 - mean1) / jnp.sqrt(var1 + 1e-5) * g1 + b1
    x1 = jnp.max(jnp.clip(hn1, 0.0, 6.0), axis=2)
    x1m = jnp.max(x1, axis=1)[:, None, :]

    sq2 = jnp.sum(x1 * x1, axis=-1)
    h2 = _knn2(x1, sq2[:, :, None], sq2[:, None, :], W2)

    mean2 = jnp.mean(h2, axis=(0, 1, 2))
    sden2 = jnp.sqrt(jnp.var(h2, axis=(0, 1, 2)) + 1e-5)

    out = pl.pallas_call(
        _finish_body,
        grid=(B,),
        in_specs=[
            pl.BlockSpec((1, N, KNN, 64), lambda b: (b, 0, 0, 0)),
            pl.BlockSpec((1, 64), lambda b: (0, 0)),
            pl.BlockSpec((1, 64), lambda b: (0, 0)),
            pl.BlockSpec((1, 64), lambda b: (0, 0)),
            pl.BlockSpec((1, 64), lambda b: (0, 0)),
            pl.BlockSpec((1, 1, 64), lambda b: (b, 0, 0)),
            pl.BlockSpec((128, 64), lambda b: (0, 0)),
            pl.BlockSpec((1, 64), lambda b: (0, 0)),
        ],
        out_specs=pl.BlockSpec((1, 1, 64), lambda b: (b, 0, 0)),
        out_shape=jax.ShapeDtypeStruct((B, 1, 64), jnp.float32),
    )(h2, mean2[None, :], sden2[None, :], g2[None, :], b2[None, :],
      x1m, Wp, bp[None, :])
    return out.reshape(B, 64)


# speedup vs baseline: 8.8258x; 1.0369x over previous
import jax
import jax.numpy as jnp
from jax.experimental import pallas as pl

B, N, KNN = 64, 1024, 4
RB = 2
R = N // RB


def _bf16_dot(a, b):
    return jax.lax.dot_general(a.astype(jnp.bfloat16), b.astype(jnp.bfloat16),
                               (((1,), (1,)), ((), ())),
                               preferred_element_type=jnp.float32)


def _select4(d, gather_from):
    iota = jax.lax.broadcasted_iota(jnp.int32, (R, N), 1)
    gs = []
    for _ in range(KNN):
        m = jnp.min(d, axis=1, keepdims=True)
        eq = d == m
        idx = jnp.min(jnp.where(eq, iota, N), axis=1, keepdims=True)
        oh = iota == idx
        g = jax.lax.dot_general(oh.astype(jnp.float32), gather_from,
                                (((1,), (0,)), ((), ())),
                                precision=jax.lax.Precision.HIGHEST,
                                preferred_element_type=jnp.float32)
        gs.append(g)
        d = jnp.where(oh, jnp.inf, d)
    return gs


def _knn1_body(xf_ref, xr_ref, sqr_ref, sqj_ref, xj_ref):
    xf = xf_ref[0]
    xr = xr_ref[0]
    d = (sqr_ref[0] + sqj_ref[0]) - 2.0 * _bf16_dot(xr, xf)
    gs = _select4(d, xf)
    for s in range(KNN):
        xj_ref[0, :, s] = gs[s]


def _knn2_body(xf_ref, xr_ref, sqr_ref, sqj_ref, w_ref, h_ref):
    xf = xf_ref[0]
    xr = xr_ref[0]
    d = (sqr_ref[0] + sqj_ref[0]) - 2.0 * _bf16_dot(xr, xf)
    gs = _select4(d, xf)
    W = w_ref[...]
    for s in range(KNN):
        edge = jnp.concatenate([xr, gs[s] - xr], axis=1)
        h_ref[0, :, s] = jax.lax.dot_general(
            edge, W, (((1,), (0,)), ((), ())),
            preferred_element_type=jnp.float32)


def _finish_body(h_ref, mn_ref, sd_ref, ga_ref, be_ref, xm_ref,
                 wp_ref, bp_ref, out_ref):
    h = h_ref[0]
    hn = ((h - mn_ref[...]) / sd_ref[...]) * ga_ref[...] + be_ref[...]
    hn = jnp.clip(hn, 0.0, 6.0)
    x2 = jnp.max(hn, axis=1)
    x2m = jnp.max(x2, axis=0, keepdims=True)
    pooled = jnp.concatenate([xm_ref[0], x2m], axis=1)
    o = jax.lax.dot_general(pooled, wp_ref[...], (((1,), (0,)), ((), ())),
                            preferred_element_type=jnp.float32)
    out_ref[0] = jnp.clip(o + bp_ref[...], 0.0, 6.0)


def _knn1(x, sqc, sqr):
    return pl.pallas_call(
        _knn1_body,
        grid=(B, RB),
        in_specs=[
            pl.BlockSpec((1, N, 3), lambda b, rb: (b, 0, 0)),
            pl.BlockSpec((1, R, 3), lambda b, rb: (b, rb, 0)),
            pl.BlockSpec((1, R, 1), lambda b, rb: (b, rb, 0)),
            pl.BlockSpec((1, 1, N), lambda b, rb: (b, 0, 0)),
        ],
        out_specs=pl.BlockSpec((1, R, KNN, 3), lambda b, rb: (b, rb, 0, 0)),
        out_shape=jax.ShapeDtypeStruct((B, N, KNN, 3), jnp.float32),
    )(x, x, sqc, sqr)


def _knn2(x1, sqc, sqr, W2):
    return pl.pallas_call(
        _knn2_body,
        grid=(B, RB),
        in_specs=[
            pl.BlockSpec((1, N, 64), lambda b, rb: (b, 0, 0)),
            pl.BlockSpec((1, R, 64), lambda b, rb: (b, rb, 0)),
            pl.BlockSpec((1, R, 1), lambda b, rb: (b, rb, 0)),
            pl.BlockSpec((1, 1, N), lambda b, rb: (b, 0, 0)),
            pl.BlockSpec((128, 64), lambda b, rb: (0, 0)),
        ],
        out_specs=pl.BlockSpec((1, R, KNN, 64), lambda b, rb: (b, rb, 0, 0)),
        out_shape=jax.ShapeDtypeStruct((B, N, KNN, 64), jnp.float32),
    )(x1, x1, sqc, sqr, W2)


@jax.jit
def kernel(x, W1, g1, b1, W2, g2, b2, Wp, bp):
    sq1 = jnp.sum(x * x, axis=-1)
    xj = _knn1(x, sq1[:, :, None], sq1[:, None, :])

    xi = jnp.broadcast_to(x[:, :, None, :], xj.shape)
    edge1 = jnp.concatenate([xi, xj - xi], axis=-1)
    h1 = edge1 @ W1
    mean1 = jnp.mean(h1, axis=(0, 1, 2))
    var1 = jnp.var(h1, axis=(0, 1, 2))
    hn1 = (h1 - mean1) / jnp.sqrt(var1 + 1e-5) * g1 + b1
    x1 = jnp.max(jnp.clip(hn1, 0.0, 6.0), axis=2)
    x1m = jnp.max(x1, axis=1)[:, None, :]

    sq2 = jnp.sum(x1 * x1, axis=-1)
    h2 = _knn2(x1, sq2[:, :, None], sq2[:, None, :], W2)

    mean2 = jnp.mean(h2, axis=(0, 1, 2))
    sden2 = jnp.sqrt(jnp.var(h2, axis=(0, 1, 2)) + 1e-5)

    out = pl.pallas_call(
        _finish_body,
        grid=(B,),
        in_specs=[
            pl.BlockSpec((1, N, KNN, 64), lambda b: (b, 0, 0, 0)),
            pl.BlockSpec((1, 64), lambda b: (0, 0)),
            pl.BlockSpec((1, 64), lambda b: (0, 0)),
            pl.BlockSpec((1, 64), lambda b: (0, 0)),
            pl.BlockSpec((1, 64), lambda b: (0, 0)),
            pl.BlockSpec((1, 1, 64), lambda b: (b, 0, 0)),
            pl.BlockSpec((128, 64), lambda b: (0, 0)),
            pl.BlockSpec((1, 64), lambda b: (0, 0)),
        ],
        out_specs=pl.BlockSpec((1, 1, 64), lambda b: (b, 0, 0)),
        out_shape=jax.ShapeDtypeStruct((B, 1, 64), jnp.float32),
    )(h2, mean2[None, :], sden2[None, :], g2[None, :], b2[None, :],
      x1m, Wp, bp[None, :])
    return out.reshape(B, 64)
